# Initial kernel scaffold; baseline (speedup 1.0000x reference)
import jax
import jax.numpy as jnp
from jax.experimental import pallas as pl


def kernel(x_author, x_paper, edge_index_author_paper, edge_index_paper_author, edge_index_paper_paper, params1, params2):
    raise NotImplementedError("write your pallas kernel here")


# speedup vs baseline: 7.8463x; 7.8463x over previous
import functools

import numpy as _np

import jax
import jax.numpy as jnp
from jax import lax
from jax.experimental import pallas as pl
from jax.experimental.pallas import tpu as pltpu
from jax.experimental.pallas import tpu_sc as plsc

N_NODE = 10000
HID = 128
HEADS = 8
D = 16
NEG = 0.2
E = 160000

NC = 2
NS = 16
L = 16
NW = NC * NS

C = 128
EPT = 5120
EPAD = NW * EPT
NCHUNK = EPT // C
RW = HID + L
NROW = 10112
RPT = NROW // NS

_HIGH = lax.Precision.HIGHEST


_BCAST_DN = lax.GatherDimensionNumbers(
    offset_dims=(), collapsed_slice_dims=(0,), start_index_map=(0,))


def _lane_bcast(v, j):
    idx = jnp.full((L, 1), j, jnp.int32)
    return lax.gather(v, idx, _BCAST_DN, (1,),
                      mode=lax.GatherScatterMode.PROMISE_IN_BOUNDS)


def _edge_body(haug_hbm, adst_hbm, esrc_hbm, edst_hbm, out_hbm,
               hv, adv, isrc, idst, acc, sem):
    c = lax.axis_index("c")
    s = lax.axis_index("s")
    wid = s * NC + c

    zero = jnp.zeros((L,), jnp.float32)

    def _zrow(i, carry):
        for j in range(RW // L):
            hv[i, pl.ds(j * L, L)] = zero
        return carry

    lax.fori_loop(0, C, _zrow, 0)
    base_row = s * RPT
    for k in range(RPT // C):
        pltpu.sync_copy(hv, acc.at[pl.ds(base_row + k * C, C)])
    rem = RPT % C
    if rem:
        pltpu.sync_copy(hv.at[pl.ds(0, rem)],
                        acc.at[pl.ds(base_row + (RPT // C) * C, rem)])
    plsc.subcore_barrier()

    ebase = wid * EPT

    def _chunk(g, carry):
        off = ebase + g * C
        pltpu.sync_copy(esrc_hbm.at[pl.ds(off, C)], isrc)
        pltpu.sync_copy(edst_hbm.at[pl.ds(off, C)], idst)
        cp1 = pltpu.async_copy(haug_hbm.at[isrc], hv, sem)
        cp2 = pltpu.async_copy(adst_hbm.at[idst], adv, sem)
        cp1.wait()
        cp2.wait()

        def _edge(e, ecarry):
            a = hv[e, pl.ds(HID, L)] + adv[e]
            a = jnp.where(a > 0, a, a * NEG)
            w = jnp.exp(a)
            hv[e, pl.ds(HID, L)] = w
            for j in range(HEADS):
                wj = _lane_bcast(w, j)
                hv[e, pl.ds(j * L, L)] = hv[e, pl.ds(j * L, L)] * wj
            return ecarry

        lax.fori_loop(0, C, _edge, 0)
        pltpu.sync_copy(hv, acc.at[idst], add=True)
        return carry

    lax.fori_loop(0, NCHUNK, _chunk, 0)
    plsc.subcore_barrier()
    pltpu.sync_copy(acc.at[pl.ds(base_row, RPT)],
                    out_hbm.at[c, pl.ds(base_row, RPT)])


_edge_pass = pl.kernel(
    _edge_body,
    out_type=jax.ShapeDtypeStruct((NC, NROW, RW), jnp.float32),
    mesh=plsc.VectorSubcoreMesh(core_axis_name="c", subcore_axis_name="s"),
    scratch_types=[
        pltpu.VMEM((C, RW), jnp.float32),
        pltpu.VMEM((C, L), jnp.float32),
        pltpu.VMEM((C,), jnp.int32),
        pltpu.VMEM((C,), jnp.int32),
        pltpu.VMEM_SHARED((NROW, RW), jnp.float32),
        pltpu.SemaphoreType.DMA,
    ],
    compiler_params=pltpu.CompilerParams(use_tc_tiling_on_sc=False),
)


def _proj_body(n_src, n_dst, x_ref, w_ref, b_ref, asrc_ref, adst_ref, *outs):
    h = jnp.dot(x_ref[...], w_ref[...], precision=_HIGH,
                preferred_element_type=jnp.float32) + b_ref[...]
    for i in range(n_src):
        outs[i][:, :HID] = h
        outs[i][:, HID:] = jnp.dot(h, asrc_ref[...][:, i * L:(i + 1) * L],
                                   precision=_HIGH)
    for j in range(n_dst):
        outs[n_src + j][...] = jnp.dot(h, adst_ref[...][:, j * L:(j + 1) * L],
                                       precision=_HIGH)


def _make_proj(n_src, n_dst):
    blk = 1000
    grid = N_NODE // blk
    full = lambda shp: pl.BlockSpec(shp, lambda i: (0,) * len(shp))
    return pl.pallas_call(
        functools.partial(_proj_body, n_src, n_dst),
        grid=(grid,),
        in_specs=[
            pl.BlockSpec((blk, HID), lambda i: (i, 0)),
            full((HID, HID)),
            full((1, HID)),
            full((HID, n_src * L)),
            full((HID, n_dst * L)),
        ],
        out_specs=[pl.BlockSpec((blk, RW), lambda i: (i, 0))] * n_src
        + [pl.BlockSpec((blk, L), lambda i: (i, 0))] * n_dst,
        out_shape=[jax.ShapeDtypeStruct((N_NODE, RW), jnp.float32)] * n_src
        + [jax.ShapeDtypeStruct((N_NODE, L), jnp.float32)] * n_dst,
    )


def _epi_body(acc_ap_ref, acc_pa_ref, acc_pp_ref, emat_ref, kw_ref, kb_ref,
              out_ap_ref, out_pa_ref, out_pp_ref, sums_ref):
    i = pl.program_id(0)
    outs = []
    for acc_ref, out_ref in ((acc_ap_ref, out_ap_ref),
                             (acc_pa_ref, out_pa_ref),
                             (acc_pp_ref, out_pp_ref)):
        t = acc_ref[0] + acc_ref[1]
        num = t[:, :HID]
        den = jnp.dot(t[:, HID:], emat_ref[...], precision=_HIGH)
        o = jnp.maximum(num / (den + 1e-16), 0.0)
        out_ref[...] = o
        outs.append(o)
    t_ap = jnp.tanh(jnp.dot(outs[0], kw_ref[...], precision=_HIGH)
                    + kb_ref[...])
    t_pp = jnp.tanh(jnp.dot(outs[2], kw_ref[...], precision=_HIGH)
                    + kb_ref[...])
    part = jnp.concatenate([t_ap.sum(0, keepdims=True),
                            t_pp.sum(0, keepdims=True)], axis=0)

    @pl.when(i == 0)
    def _():
        sums_ref[...] = jnp.zeros_like(sums_ref)

    sums_ref[...] += part


def _make_epi():
    blk = 1000
    grid = N_NODE // blk
    acc_spec = pl.BlockSpec((NC, blk, RW), lambda i: (0, i, 0))
    full = lambda shp: pl.BlockSpec(shp, lambda i: (0,) * len(shp))
    return pl.pallas_call(
        _epi_body,
        grid=(grid,),
        in_specs=[acc_spec, acc_spec, acc_spec,
                  full((L, HID)), full((HID, HID)), full((1, HID))],
        out_specs=[pl.BlockSpec((blk, HID), lambda i: (i, 0))] * 3
        + [full((2, HID))],
        out_shape=[jax.ShapeDtypeStruct((N_NODE, HID), jnp.float32)] * 3
        + [jax.ShapeDtypeStruct((2, HID), jnp.float32)],
    )


def _mix_body(x_ref, y_ref, attn_ref, out_ref):
    out_ref[...] = jnp.maximum(
        attn_ref[0] * x_ref[...] + attn_ref[1] * y_ref[...], 0.0)


def _make_mix():
    blk = 1000
    grid = N_NODE // blk
    return pl.pallas_call(
        _mix_body,
        grid=(grid,),
        in_specs=[pl.BlockSpec((blk, HID), lambda i: (i, 0)),
                  pl.BlockSpec((blk, HID), lambda i: (i, 0)),
                  pl.BlockSpec(memory_space=pltpu.SMEM)],
        out_specs=pl.BlockSpec((blk, HID), lambda i: (i, 0)),
        out_shape=jax.ShapeDtypeStruct((N_NODE, HID), jnp.float32),
    )


_proj_a = _make_proj(1, 1)
_proj_p = _make_proj(2, 2)
_epi = _make_epi()
_mix = _make_mix()


def _att_mat(att):
    sel = _np.zeros((HID, L), _np.float32)
    for h in range(HEADS):
        sel[h * D:(h + 1) * D, h] = 1.0
    return jnp.tile(att.reshape(HID)[:, None], (1, L)) * sel


def _emat():
    m = _np.zeros((L, HID), _np.float32)
    for h in range(HEADS):
        m[h, h * D:(h + 1) * D] = 1.0
    return m


def _pad_edges(ei):
    npad = EPAD - E
    esrc = jnp.concatenate([ei[0], jnp.zeros((npad,), jnp.int32)])
    edst = jnp.concatenate([ei[1], jnp.full((npad,), N_NODE, jnp.int32)])
    return esrc, edst


def _layer(x_a, x_p, edges, p):
    (esrc_ap, edst_ap), (esrc_pa, edst_pa), (esrc_pp, edst_pp) = edges
    b_a = p['proj_author_b'].reshape(1, HID)
    b_p = p['proj_paper_b'].reshape(1, HID)
    asrc_a = _att_mat(p['att_src_ap'])
    adst_a = _att_mat(p['att_dst_pa'])
    asrc_p = jnp.concatenate(
        [_att_mat(p['att_src_pa']), _att_mat(p['att_src_pp'])], axis=1)
    adst_p = jnp.concatenate(
        [_att_mat(p['att_dst_ap']), _att_mat(p['att_dst_pp'])], axis=1)

    haug_ap, adst_pa16 = _proj_a(x_a, p['proj_author_W'], b_a, asrc_a, adst_a)
    haug_pa, haug_pp, adst_ap16, adst_pp16 = _proj_p(
        x_p, p['proj_paper_W'], b_p, asrc_p, adst_p)

    acc_ap = _edge_pass(haug_ap, adst_ap16, esrc_ap, edst_ap)
    acc_pa = _edge_pass(haug_pa, adst_pa16, esrc_pa, edst_pa)
    acc_pp = _edge_pass(haug_pp, adst_pp16, esrc_pp, edst_pp)

    out_ap, out_pa, out_pp, sums = _epi(
        acc_ap, acc_pa, acc_pp, jnp.asarray(_emat()), p['k_lin_W'],
        p['k_lin_b'].reshape(1, HID))

    score = (p['q'][None, :] * (sums / N_NODE)).sum(-1)
    attn = jax.nn.softmax(score)
    p_out = _mix(out_ap, out_pp, attn)
    return out_pa, p_out


def kernel(x_author, x_paper, edge_index_author_paper,
           edge_index_paper_author, edge_index_paper_paper,
           params1, params2):
    edges = (_pad_edges(edge_index_author_paper),
             _pad_edges(edge_index_paper_author),
             _pad_edges(edge_index_paper_paper))
    a1, p1 = _layer(x_author, x_paper, edges, params1)
    a2, p2 = _layer(a1, p1, edges, params2)
    return a2, p2


# speedup vs baseline: 11.7201x; 1.4937x over previous
import functools

import numpy as _np

import jax
import jax.numpy as jnp
from jax import lax
from jax.experimental import pallas as pl
from jax.experimental.pallas import tpu as pltpu
from jax.experimental.pallas import tpu_sc as plsc

N_NODE = 10000
HID = 128
HEADS = 8
D = 16
NEG = 0.2
E = 160000

NC = 2
NS = 16
L = 16
NW = NC * NS

C = 64
EPT = 5120
EPAD = NW * EPT
NCHUNK = EPT // C
RW = HID + L
NROW = 10112
RPT = NROW // NS

_HIGH = lax.Precision.HIGHEST


_BCAST_DN = lax.GatherDimensionNumbers(
    offset_dims=(), collapsed_slice_dims=(0,), start_index_map=(0,))


def _lane_bcast(v, j):
    idx = jnp.full((L, 1), j, jnp.int32)
    return lax.gather(v, idx, _BCAST_DN, (1,),
                      mode=lax.GatherScatterMode.PROMISE_IN_BOUNDS)


def _edge_body(haug_hbm, adst_hbm, esrc_hbm, edst_hbm, out_hbm,
               hv0, hv1, adv0, adv1, isrc, idst, acc,
               sg0, sg1, ss0, ss1):
    c = lax.axis_index("c")
    s = lax.axis_index("s")
    wid = s * NC + c
    hv = (hv0, hv1)
    adv = (adv0, adv1)
    sg = (sg0, sg1)
    ss = (ss0, ss1)

    zero = jnp.zeros((L,), jnp.float32)

    def _zrow(i, carry):
        for j in range(RW // L):
            hv0[i, pl.ds(j * L, L)] = zero
        return carry

    lax.fori_loop(0, C, _zrow, 0)
    base_row = s * RPT
    for k in range(RPT // C):
        pltpu.sync_copy(hv0, acc.at[pl.ds(base_row + k * C, C)])
    rem = RPT % C
    if rem:
        pltpu.sync_copy(hv0.at[pl.ds(0, rem)],
                        acc.at[pl.ds(base_row + (RPT // C) * C, rem)])
    plsc.subcore_barrier()

    rbase = wid * NCHUNK
    pltpu.sync_copy(esrc_hbm.at[pl.ds(rbase, NCHUNK)], isrc)
    pltpu.sync_copy(edst_hbm.at[pl.ds(rbase, NCHUNK)], idst)

    def _issue_gather(g, b):
        pltpu.async_copy(haug_hbm.at[isrc.at[g]], hv[b], sg[b])
        pltpu.async_copy(adst_hbm.at[idst.at[g]], adv[b], sg[b])

    def _wait_gather(b):
        pltpu.make_async_copy(haug_hbm.at[pl.ds(0, C)], hv[b], sg[b]).wait()
        pltpu.make_async_copy(adst_hbm.at[pl.ds(0, C)], adv[b], sg[b]).wait()

    def _compute(b):
        hb, ab = hv[b], adv[b]

        @plsc.parallel_loop(0, C, unroll=2)
        def _edge(e):
            a = hb[e, pl.ds(HID, L)] + ab[e]
            a = jnp.maximum(a, a * NEG)
            w = jnp.exp(a)
            hb[e, pl.ds(HID, L)] = w
            for j in range(HEADS):
                wj = _lane_bcast(w, j)
                hb[e, pl.ds(j * L, L)] = hb[e, pl.ds(j * L, L)] * wj

    def _slot(g, b, prefetch):
        _wait_gather(b)
        _compute(b)
        cp = pltpu.async_copy(hv[b], acc.at[idst.at[g]], ss[b], add=True)
        cp.wait()
        if prefetch:
            _issue_gather(g + 2, b)

    _issue_gather(0, 0)
    _issue_gather(1, 1)

    def _pair(p, carry):
        for b in (0, 1):
            _slot(2 * p + b, b, True)
        return carry

    lax.fori_loop(0, NCHUNK // 2 - 1, _pair, 0)
    _slot(NCHUNK - 2, 0, False)
    _slot(NCHUNK - 1, 1, False)

    plsc.subcore_barrier()
    pltpu.sync_copy(acc.at[pl.ds(base_row, RPT)],
                    out_hbm.at[c, pl.ds(base_row, RPT)])


_edge_pass = pl.kernel(
    _edge_body,
    out_type=jax.ShapeDtypeStruct((NC, NROW, RW), jnp.float32),
    mesh=plsc.VectorSubcoreMesh(core_axis_name="c", subcore_axis_name="s"),
    scratch_types=[
        pltpu.VMEM((C, RW), jnp.float32),
        pltpu.VMEM((C, RW), jnp.float32),
        pltpu.VMEM((C, L), jnp.float32),
        pltpu.VMEM((C, L), jnp.float32),
        pltpu.VMEM((NCHUNK, C), jnp.int32),
        pltpu.VMEM((NCHUNK, C), jnp.int32),
        pltpu.VMEM_SHARED((NROW, RW), jnp.float32),
        pltpu.SemaphoreType.DMA,
        pltpu.SemaphoreType.DMA,
        pltpu.SemaphoreType.DMA,
        pltpu.SemaphoreType.DMA,
    ],
    compiler_params=pltpu.CompilerParams(use_tc_tiling_on_sc=False),
)


def _proj_body(n_src, n_dst, x_ref, w_ref, b_ref, asrc_ref, adst_ref, *outs):
    h = jnp.dot(x_ref[...], w_ref[...], precision=_HIGH,
                preferred_element_type=jnp.float32) + b_ref[...]
    for i in range(n_src):
        outs[i][:, :HID] = h
        outs[i][:, HID:] = jnp.dot(h, asrc_ref[...][:, i * L:(i + 1) * L],
                                   precision=_HIGH)
    for j in range(n_dst):
        outs[n_src + j][...] = jnp.dot(h, adst_ref[...][:, j * L:(j + 1) * L],
                                       precision=_HIGH)


def _make_proj(n_src, n_dst):
    blk = 1000
    grid = N_NODE // blk
    full = lambda shp: pl.BlockSpec(shp, lambda i: (0,) * len(shp))
    return pl.pallas_call(
        functools.partial(_proj_body, n_src, n_dst),
        grid=(grid,),
        in_specs=[
            pl.BlockSpec((blk, HID), lambda i: (i, 0)),
            full((HID, HID)),
            full((1, HID)),
            full((HID, n_src * L)),
            full((HID, n_dst * L)),
        ],
        out_specs=[pl.BlockSpec((blk, RW), lambda i: (i, 0))] * n_src
        + [pl.BlockSpec((blk, L), lambda i: (i, 0))] * n_dst,
        out_shape=[jax.ShapeDtypeStruct((N_NODE, RW), jnp.float32)] * n_src
        + [jax.ShapeDtypeStruct((N_NODE, L), jnp.float32)] * n_dst,
    )


def _epi_body(acc_ap_ref, acc_pa_ref, acc_pp_ref, emat_ref, kw_ref, kb_ref,
              out_ap_ref, out_pa_ref, out_pp_ref, sums_ref):
    i = pl.program_id(0)
    outs = []
    for acc_ref, out_ref in ((acc_ap_ref, out_ap_ref),
                             (acc_pa_ref, out_pa_ref),
                             (acc_pp_ref, out_pp_ref)):
        t = acc_ref[0] + acc_ref[1]
        num = t[:, :HID]
        den = jnp.dot(t[:, HID:], emat_ref[...], precision=_HIGH)
        o = jnp.maximum(num / (den + 1e-16), 0.0)
        out_ref[...] = o
        outs.append(o)
    t_ap = jnp.tanh(jnp.dot(outs[0], kw_ref[...], precision=_HIGH)
                    + kb_ref[...])
    t_pp = jnp.tanh(jnp.dot(outs[2], kw_ref[...], precision=_HIGH)
                    + kb_ref[...])
    part = jnp.concatenate([t_ap.sum(0, keepdims=True),
                            t_pp.sum(0, keepdims=True)], axis=0)

    @pl.when(i == 0)
    def _():
        sums_ref[...] = jnp.zeros_like(sums_ref)

    sums_ref[...] += part


def _make_epi():
    blk = 1000
    grid = N_NODE // blk
    acc_spec = pl.BlockSpec((NC, blk, RW), lambda i: (0, i, 0))
    full = lambda shp: pl.BlockSpec(shp, lambda i: (0,) * len(shp))
    return pl.pallas_call(
        _epi_body,
        grid=(grid,),
        in_specs=[acc_spec, acc_spec, acc_spec,
                  full((L, HID)), full((HID, HID)), full((1, HID))],
        out_specs=[pl.BlockSpec((blk, HID), lambda i: (i, 0))] * 3
        + [full((2, HID))],
        out_shape=[jax.ShapeDtypeStruct((N_NODE, HID), jnp.float32)] * 3
        + [jax.ShapeDtypeStruct((2, HID), jnp.float32)],
    )


def _mix_body(x_ref, y_ref, attn_ref, out_ref):
    out_ref[...] = jnp.maximum(
        attn_ref[0] * x_ref[...] + attn_ref[1] * y_ref[...], 0.0)


def _make_mix():
    blk = 1000
    grid = N_NODE // blk
    return pl.pallas_call(
        _mix_body,
        grid=(grid,),
        in_specs=[pl.BlockSpec((blk, HID), lambda i: (i, 0)),
                  pl.BlockSpec((blk, HID), lambda i: (i, 0)),
                  pl.BlockSpec(memory_space=pltpu.SMEM)],
        out_specs=pl.BlockSpec((blk, HID), lambda i: (i, 0)),
        out_shape=jax.ShapeDtypeStruct((N_NODE, HID), jnp.float32),
    )


_proj_a = _make_proj(1, 1)
_proj_p = _make_proj(2, 2)
_epi = _make_epi()
_mix = _make_mix()


def _att_mat(att):
    sel = _np.zeros((HID, L), _np.float32)
    for h in range(HEADS):
        sel[h * D:(h + 1) * D, h] = 1.0
    return jnp.tile(att.reshape(HID)[:, None], (1, L)) * sel


def _emat():
    m = _np.zeros((L, HID), _np.float32)
    for h in range(HEADS):
        m[h, h * D:(h + 1) * D] = 1.0
    return m


def _pad_edges(ei):
    npad = EPAD - E
    esrc = jnp.concatenate(
        [ei[0], jnp.zeros((npad,), jnp.int32)]).reshape(NW * NCHUNK, C)
    edst = jnp.concatenate(
        [ei[1], jnp.full((npad,), N_NODE, jnp.int32)]).reshape(NW * NCHUNK, C)
    return esrc, edst


def _layer(x_a, x_p, edges, p):
    (esrc_ap, edst_ap), (esrc_pa, edst_pa), (esrc_pp, edst_pp) = edges
    b_a = p['proj_author_b'].reshape(1, HID)
    b_p = p['proj_paper_b'].reshape(1, HID)
    asrc_a = _att_mat(p['att_src_ap'])
    adst_a = _att_mat(p['att_dst_pa'])
    asrc_p = jnp.concatenate(
        [_att_mat(p['att_src_pa']), _att_mat(p['att_src_pp'])], axis=1)
    adst_p = jnp.concatenate(
        [_att_mat(p['att_dst_ap']), _att_mat(p['att_dst_pp'])], axis=1)

    haug_ap, adst_pa16 = _proj_a(x_a, p['proj_author_W'], b_a, asrc_a, adst_a)
    haug_pa, haug_pp, adst_ap16, adst_pp16 = _proj_p(
        x_p, p['proj_paper_W'], b_p, asrc_p, adst_p)

    acc_ap = _edge_pass(haug_ap, adst_ap16, esrc_ap, edst_ap)
    acc_pa = _edge_pass(haug_pa, adst_pa16, esrc_pa, edst_pa)
    acc_pp = _edge_pass(haug_pp, adst_pp16, esrc_pp, edst_pp)

    out_ap, out_pa, out_pp, sums = _epi(
        acc_ap, acc_pa, acc_pp, jnp.asarray(_emat()), p['k_lin_W'],
        p['k_lin_b'].reshape(1, HID))

    score = (p['q'][None, :] * (sums / N_NODE)).sum(-1)
    attn = jax.nn.softmax(score)
    p_out = _mix(out_ap, out_pp, attn)
    return out_pa, p_out


def kernel(x_author, x_paper, edge_index_author_paper,
           edge_index_paper_author, edge_index_paper_paper,
           params1, params2):
    edges = (_pad_edges(edge_index_author_paper),
             _pad_edges(edge_index_paper_author),
             _pad_edges(edge_index_paper_paper))
    a1, p1 = _layer(x_author, x_paper, edges, params1)
    a2, p2 = _layer(a1, p1, edges, params2)
    return a2, p2


# speedup vs baseline: 11.7202x; 1.0000x over previous
import functools

import numpy as _np

import jax
import jax.numpy as jnp
from jax import lax
from jax.experimental import pallas as pl
from jax.experimental.pallas import tpu as pltpu
from jax.experimental.pallas import tpu_sc as plsc

N_NODE = 10000
HID = 128
HEADS = 8
D = 16
NEG = 0.2
E = 160000

NC = 2
NS = 16
L = 16
NW = NC * NS

C = 64
EPT = 5120
EPAD = NW * EPT
NCHUNK = EPT // C
RW = HID + L
NROW = 10112
RPT = NROW // NS

_HIGH = lax.Precision.HIGHEST


_BCAST_DN = lax.GatherDimensionNumbers(
    offset_dims=(), collapsed_slice_dims=(0,), start_index_map=(0,))


def _lane_bcast(v, j):
    idx = jnp.full((L, 1), j, jnp.int32)
    return lax.gather(v, idx, _BCAST_DN, (1,),
                      mode=lax.GatherScatterMode.PROMISE_IN_BOUNDS)


def _edge_body(haug_hbm, adst_hbm, esrc_hbm, edst_hbm, out_hbm,
               hv0, hv1, adv0, adv1, isrc, idst, acc,
               sg0, sg1, ss0, ss1):
    c = lax.axis_index("c")
    s = lax.axis_index("s")
    wid = s * NC + c
    hv = (hv0, hv1)
    adv = (adv0, adv1)
    sg = (sg0, sg1)
    ss = (ss0, ss1)

    zero = jnp.zeros((L,), jnp.float32)

    def _zrow(i, carry):
        for j in range(RW // L):
            hv0[i, pl.ds(j * L, L)] = zero
        return carry

    lax.fori_loop(0, C, _zrow, 0)
    base_row = s * RPT
    for k in range(RPT // C):
        pltpu.sync_copy(hv0, acc.at[pl.ds(base_row + k * C, C)])
    rem = RPT % C
    if rem:
        pltpu.sync_copy(hv0.at[pl.ds(0, rem)],
                        acc.at[pl.ds(base_row + (RPT // C) * C, rem)])
    plsc.subcore_barrier()

    rbase = wid * NCHUNK
    pltpu.sync_copy(esrc_hbm.at[pl.ds(rbase, NCHUNK)], isrc)
    pltpu.sync_copy(edst_hbm.at[pl.ds(rbase, NCHUNK)], idst)

    def _issue_gather(g, b):
        pltpu.async_copy(haug_hbm.at[isrc.at[g]], hv[b], sg[b])
        pltpu.async_copy(adst_hbm.at[idst.at[g]], adv[b], sg[b])

    def _wait_gather(b):
        pltpu.make_async_copy(haug_hbm.at[pl.ds(0, C)], hv[b], sg[b]).wait()
        pltpu.make_async_copy(adst_hbm.at[pl.ds(0, C)], adv[b], sg[b]).wait()

    def _compute(b):
        hb, ab = hv[b], adv[b]

        @plsc.parallel_loop(0, C, unroll=4)
        def _edge(e):
            a = hb[e, pl.ds(HID, L)] + ab[e]
            a = jnp.maximum(a, a * NEG)
            w = jnp.exp(a)
            hb[e, pl.ds(HID, L)] = w
            for j in range(HEADS):
                wj = _lane_bcast(w, j)
                hb[e, pl.ds(j * L, L)] = hb[e, pl.ds(j * L, L)] * wj

    def _slot(g, b, prefetch):
        _wait_gather(b)
        _compute(b)
        cp = pltpu.async_copy(hv[b], acc.at[idst.at[g]], ss[b], add=True)
        cp.wait()
        if prefetch:
            _issue_gather(g + 2, b)

    _issue_gather(0, 0)
    _issue_gather(1, 1)

    def _pair(p, carry):
        for b in (0, 1):
            _slot(2 * p + b, b, True)
        return carry

    lax.fori_loop(0, NCHUNK // 2 - 1, _pair, 0)
    _slot(NCHUNK - 2, 0, False)
    _slot(NCHUNK - 1, 1, False)

    plsc.subcore_barrier()
    pltpu.sync_copy(acc.at[pl.ds(base_row, RPT)],
                    out_hbm.at[c, pl.ds(base_row, RPT)])


_edge_pass = pl.kernel(
    _edge_body,
    out_type=jax.ShapeDtypeStruct((NC, NROW, RW), jnp.float32),
    mesh=plsc.VectorSubcoreMesh(core_axis_name="c", subcore_axis_name="s"),
    scratch_types=[
        pltpu.VMEM((C, RW), jnp.float32),
        pltpu.VMEM((C, RW), jnp.float32),
        pltpu.VMEM((C, L), jnp.float32),
        pltpu.VMEM((C, L), jnp.float32),
        pltpu.VMEM((NCHUNK, C), jnp.int32),
        pltpu.VMEM((NCHUNK, C), jnp.int32),
        pltpu.VMEM_SHARED((NROW, RW), jnp.float32),
        pltpu.SemaphoreType.DMA,
        pltpu.SemaphoreType.DMA,
        pltpu.SemaphoreType.DMA,
        pltpu.SemaphoreType.DMA,
    ],
    compiler_params=pltpu.CompilerParams(use_tc_tiling_on_sc=False),
)


def _proj_body(n_src, n_dst, x_ref, w_ref, b_ref, asrc_ref, adst_ref, *outs):
    h = jnp.dot(x_ref[...], w_ref[...], precision=_HIGH,
                preferred_element_type=jnp.float32) + b_ref[...]
    for i in range(n_src):
        outs[i][:, :HID] = h
        outs[i][:, HID:] = jnp.dot(h, asrc_ref[...][:, i * L:(i + 1) * L],
                                   precision=_HIGH)
    for j in range(n_dst):
        outs[n_src + j][...] = jnp.dot(h, adst_ref[...][:, j * L:(j + 1) * L],
                                       precision=_HIGH)


def _make_proj(n_src, n_dst):
    blk = 1000
    grid = N_NODE // blk
    full = lambda shp: pl.BlockSpec(shp, lambda i: (0,) * len(shp))
    return pl.pallas_call(
        functools.partial(_proj_body, n_src, n_dst),
        grid=(grid,),
        in_specs=[
            pl.BlockSpec((blk, HID), lambda i: (i, 0)),
            full((HID, HID)),
            full((1, HID)),
            full((HID, n_src * L)),
            full((HID, n_dst * L)),
        ],
        out_specs=[pl.BlockSpec((blk, RW), lambda i: (i, 0))] * n_src
        + [pl.BlockSpec((blk, L), lambda i: (i, 0))] * n_dst,
        out_shape=[jax.ShapeDtypeStruct((N_NODE, RW), jnp.float32)] * n_src
        + [jax.ShapeDtypeStruct((N_NODE, L), jnp.float32)] * n_dst,
    )


def _epi_body(acc_ap_ref, acc_pa_ref, acc_pp_ref, emat_ref, kw_ref, kb_ref,
              out_ap_ref, out_pa_ref, out_pp_ref, sums_ref):
    i = pl.program_id(0)
    outs = []
    for acc_ref, out_ref in ((acc_ap_ref, out_ap_ref),
                             (acc_pa_ref, out_pa_ref),
                             (acc_pp_ref, out_pp_ref)):
        t = acc_ref[0] + acc_ref[1]
        num = t[:, :HID]
        den = jnp.dot(t[:, HID:], emat_ref[...], precision=_HIGH)
        o = jnp.maximum(num / (den + 1e-16), 0.0)
        out_ref[...] = o
        outs.append(o)
    t_ap = jnp.tanh(jnp.dot(outs[0], kw_ref[...], precision=_HIGH)
                    + kb_ref[...])
    t_pp = jnp.tanh(jnp.dot(outs[2], kw_ref[...], precision=_HIGH)
                    + kb_ref[...])
    part = jnp.concatenate([t_ap.sum(0, keepdims=True),
                            t_pp.sum(0, keepdims=True)], axis=0)

    @pl.when(i == 0)
    def _():
        sums_ref[...] = jnp.zeros_like(sums_ref)

    sums_ref[...] += part


def _make_epi():
    blk = 1000
    grid = N_NODE // blk
    acc_spec = pl.BlockSpec((NC, blk, RW), lambda i: (0, i, 0))
    full = lambda shp: pl.BlockSpec(shp, lambda i: (0,) * len(shp))
    return pl.pallas_call(
        _epi_body,
        grid=(grid,),
        in_specs=[acc_spec, acc_spec, acc_spec,
                  full((L, HID)), full((HID, HID)), full((1, HID))],
        out_specs=[pl.BlockSpec((blk, HID), lambda i: (i, 0))] * 3
        + [full((2, HID))],
        out_shape=[jax.ShapeDtypeStruct((N_NODE, HID), jnp.float32)] * 3
        + [jax.ShapeDtypeStruct((2, HID), jnp.float32)],
    )


def _mix_body(x_ref, y_ref, attn_ref, out_ref):
    out_ref[...] = jnp.maximum(
        attn_ref[0] * x_ref[...] + attn_ref[1] * y_ref[...], 0.0)


def _make_mix():
    blk = 1000
    grid = N_NODE // blk
    return pl.pallas_call(
        _mix_body,
        grid=(grid,),
        in_specs=[pl.BlockSpec((blk, HID), lambda i: (i, 0)),
                  pl.BlockSpec((blk, HID), lambda i: (i, 0)),
                  pl.BlockSpec(memory_space=pltpu.SMEM)],
        out_specs=pl.BlockSpec((blk, HID), lambda i: (i, 0)),
        out_shape=jax.ShapeDtypeStruct((N_NODE, HID), jnp.float32),
    )


_proj_a = _make_proj(1, 1)
_proj_p = _make_proj(2, 2)
_epi = _make_epi()
_mix = _make_mix()


def _att_mat(att):
    sel = _np.zeros((HID, L), _np.float32)
    for h in range(HEADS):
        sel[h * D:(h + 1) * D, h] = 1.0
    return jnp.tile(att.reshape(HID)[:, None], (1, L)) * sel


def _emat():
    m = _np.zeros((L, HID), _np.float32)
    for h in range(HEADS):
        m[h, h * D:(h + 1) * D] = 1.0
    return m


def _pad_edges(ei):
    npad = EPAD - E
    esrc = jnp.concatenate(
        [ei[0], jnp.zeros((npad,), jnp.int32)]).reshape(NW * NCHUNK, C)
    edst = jnp.concatenate(
        [ei[1], jnp.full((npad,), N_NODE, jnp.int32)]).reshape(NW * NCHUNK, C)
    return esrc, edst


def _layer(x_a, x_p, edges, p):
    (esrc_ap, edst_ap), (esrc_pa, edst_pa), (esrc_pp, edst_pp) = edges
    b_a = p['proj_author_b'].reshape(1, HID)
    b_p = p['proj_paper_b'].reshape(1, HID)
    asrc_a = _att_mat(p['att_src_ap'])
    adst_a = _att_mat(p['att_dst_pa'])
    asrc_p = jnp.concatenate(
        [_att_mat(p['att_src_pa']), _att_mat(p['att_src_pp'])], axis=1)
    adst_p = jnp.concatenate(
        [_att_mat(p['att_dst_ap']), _att_mat(p['att_dst_pp'])], axis=1)

    haug_ap, adst_pa16 = _proj_a(x_a, p['proj_author_W'], b_a, asrc_a, adst_a)
    haug_pa, haug_pp, adst_ap16, adst_pp16 = _proj_p(
        x_p, p['proj_paper_W'], b_p, asrc_p, adst_p)

    acc_ap = _edge_pass(haug_ap, adst_ap16, esrc_ap, edst_ap)
    acc_pa = _edge_pass(haug_pa, adst_pa16, esrc_pa, edst_pa)
    acc_pp = _edge_pass(haug_pp, adst_pp16, esrc_pp, edst_pp)

    out_ap, out_pa, out_pp, sums = _epi(
        acc_ap, acc_pa, acc_pp, jnp.asarray(_emat()), p['k_lin_W'],
        p['k_lin_b'].reshape(1, HID))

    score = (p['q'][None, :] * (sums / N_NODE)).sum(-1)
    attn = jax.nn.softmax(score)
    p_out = _mix(out_ap, out_pp, attn)
    return out_pa, p_out


def kernel(x_author, x_paper, edge_index_author_paper,
           edge_index_paper_author, edge_index_paper_paper,
           params1, params2):
    edges = (_pad_edges(edge_index_author_paper),
             _pad_edges(edge_index_paper_author),
             _pad_edges(edge_index_paper_paper))
    a1, p1 = _layer(x_author, x_paper, edges, params1)
    a2, p2 = _layer(a1, p1, edges, params2)
    return a2, p2


# speedup vs baseline: 13.4768x; 1.1499x over previous
import functools

import numpy as _np

import jax
import jax.numpy as jnp
from jax import lax
from jax.experimental import pallas as pl
from jax.experimental.pallas import tpu as pltpu
from jax.experimental.pallas import tpu_sc as plsc

N_NODE = 10000
HID = 128
HEADS = 8
D = 16
NEG = 0.2
E = 160000

NC = 2
NS = 16
L = 16
NW = NC * NS

C = 64
EPT = 5120
EPAD = NW * EPT
NCHUNK = EPT // C
RW = HID + L
TW = 160
NROW = 10008
RPT = 632
RPT_LAST = NROW - 15 * RPT

_HIGH = lax.Precision.HIGHEST


_BCAST_DN = lax.GatherDimensionNumbers(
    offset_dims=(), collapsed_slice_dims=(0,), start_index_map=(0,))


def _lane_bcast(v, j):
    idx = jnp.full((L, 1), j, jnp.int32)
    return lax.gather(v, idx, _BCAST_DN, (1,),
                      mode=lax.GatherScatterMode.PROMISE_IN_BOUNDS)


def _edge_body(haug_hbm, adst_hbm, esrc_hbm, edst_hbm, out_hbm,
               hv0, hv1, sv0, sv1, adv0, adv1, isrc, idst, acc,
               sg0, sg1, ss0, ss1):
    c = lax.axis_index("c")
    s = lax.axis_index("s")
    wid = s * NC + c
    hv = (hv0, hv1)
    sv = (sv0, sv1)
    adv = (adv0, adv1)
    sg = (sg0, sg1)
    ss = (ss0, ss1)

    zero = jnp.zeros((L,), jnp.float32)

    def _zrow(i, carry):
        for j in range(RW // L):
            sv0[i, pl.ds(j * L, L)] = zero
        return carry

    lax.fori_loop(0, C, _zrow, 0)
    base_row = s * RPT

    def _zcopy(k, carry):
        pltpu.sync_copy(sv0, acc.at[pl.ds(base_row + k * C, C)])
        return carry

    lax.fori_loop(0, 512 // C, _zcopy, 0)

    @pl.when(s < NS - 1)
    def _():
        pltpu.sync_copy(sv0, acc.at[pl.ds(base_row + 512, C)])
        pltpu.sync_copy(sv0.at[pl.ds(0, RPT - 576)],
                        acc.at[pl.ds(base_row + 576, RPT - 576)])

    @pl.when(s == NS - 1)
    def _():
        pltpu.sync_copy(sv0.at[pl.ds(0, RPT_LAST - 512)],
                        acc.at[pl.ds(base_row + 512, RPT_LAST - 512)])
    plsc.subcore_barrier()

    rbase = wid * NCHUNK
    pltpu.sync_copy(esrc_hbm.at[pl.ds(rbase, NCHUNK)], isrc)
    pltpu.sync_copy(edst_hbm.at[pl.ds(rbase, NCHUNK)], idst)

    def _issue_gather(g, b):
        pltpu.async_copy(haug_hbm.at[isrc.at[g]], hv[b], sg[b])
        pltpu.async_copy(adst_hbm.at[idst.at[g]], adv[b], sg[b])

    def _wait_gather(b):
        pltpu.make_async_copy(haug_hbm.at[pl.ds(0, C)], hv[b], sg[b]).wait()
        pltpu.make_async_copy(adst_hbm.at[pl.ds(0, C)], adv[b], sg[b]).wait()

    def _wait_scatter(b):
        pltpu.make_async_copy(sv[b], acc.at[pl.ds(0, C)], ss[b]).wait()

    _MASK = jnp.int32(-65536)

    def _lo(v):
        return lax.bitcast_convert_type(lax.shift_left(v, 16), jnp.float32)

    def _hi(v):
        return lax.bitcast_convert_type(lax.bitwise_and(v, _MASK),
                                        jnp.float32)

    def _compute(b):
        hb, sb, ab = hv[b], sv[b], adv[b]

        @plsc.parallel_loop(0, C, unroll=4)
        def _edge(e):
            a = _lo(hb[e, pl.ds(4 * L, L)]) + ab[e]
            a = jnp.maximum(a, a * NEG)
            w = jnp.exp(a)
            sb[e, pl.ds(HID, L)] = w
            for j in range(HEADS // 2):
                v = hb[e, pl.ds(L * j, L)]
                sb[e, pl.ds(2 * L * j, L)] = _lo(v) * _lane_bcast(w, 2 * j)
                sb[e, pl.ds(2 * L * j + L, L)] = (
                    _hi(v) * _lane_bcast(w, 2 * j + 1))

    def _slot(g, b, first, prefetch):
        _wait_gather(b)
        if not first:
            _wait_scatter(b)
        _compute(b)
        pltpu.async_copy(sv[b], acc.at[idst.at[g]], ss[b], add=True)
        if prefetch:
            _issue_gather(g + 2, b)

    _issue_gather(0, 0)
    _issue_gather(1, 1)
    _slot(0, 0, True, True)
    _slot(1, 1, True, True)

    def _pair(p, carry):
        for b in (0, 1):
            _slot(2 * p + b, b, False, True)
        return carry

    lax.fori_loop(1, NCHUNK // 2 - 1, _pair, 0)
    _slot(NCHUNK - 2, 0, False, False)
    _slot(NCHUNK - 1, 1, False, False)
    _wait_scatter(0)
    _wait_scatter(1)

    plsc.subcore_barrier()

    @pl.when(s < NS - 1)
    def _():
        pltpu.sync_copy(acc.at[pl.ds(base_row, RPT)],
                        out_hbm.at[c, pl.ds(base_row, RPT)])

    @pl.when(s == NS - 1)
    def _():
        pltpu.sync_copy(acc.at[pl.ds((NS - 1) * RPT, RPT_LAST)],
                        out_hbm.at[c, pl.ds((NS - 1) * RPT, RPT_LAST)])


_edge_pass = pl.kernel(
    _edge_body,
    out_type=jax.ShapeDtypeStruct((NC, NROW, RW), jnp.float32),
    mesh=plsc.VectorSubcoreMesh(core_axis_name="c", subcore_axis_name="s"),
    scratch_types=[
        pltpu.VMEM((C, TW // 2), jnp.int32),
        pltpu.VMEM((C, TW // 2), jnp.int32),
        pltpu.VMEM((C, RW), jnp.float32),
        pltpu.VMEM((C, RW), jnp.float32),
        pltpu.VMEM((C, L), jnp.float32),
        pltpu.VMEM((C, L), jnp.float32),
        pltpu.VMEM((NCHUNK, C), jnp.int32),
        pltpu.VMEM((NCHUNK, C), jnp.int32),
        pltpu.VMEM_SHARED((NROW, RW), jnp.float32),
        pltpu.SemaphoreType.DMA,
        pltpu.SemaphoreType.DMA,
        pltpu.SemaphoreType.DMA,
        pltpu.SemaphoreType.DMA,
    ],
    compiler_params=pltpu.CompilerParams(use_tc_tiling_on_sc=False),
)


def _proj_body(n_src, n_dst, x_ref, w_ref, b_ref, asrc_ref, adst_ref,
               pmat_ref, smat_ref, *outs):
    h = jnp.dot(x_ref[...], w_ref[...], precision=_HIGH,
                preferred_element_type=jnp.float32) + b_ref[...]
    hperm = jnp.dot(h, pmat_ref[...], precision=_HIGH)
    for i in range(n_src):
        a16 = jnp.dot(h, asrc_ref[...][:, i * L:(i + 1) * L], precision=_HIGH)
        outs[i][:, :HID] = hperm.astype(jnp.bfloat16)
        outs[i][:, HID:] = jnp.dot(a16, smat_ref[...],
                                   precision=_HIGH).astype(jnp.bfloat16)
    for j in range(n_dst):
        outs[n_src + j][...] = jnp.dot(h, adst_ref[...][:, j * L:(j + 1) * L],
                                       precision=_HIGH)


def _make_proj(n_src, n_dst):
    blk = 1000
    grid = N_NODE // blk
    full = lambda shp: pl.BlockSpec(shp, lambda i: (0,) * len(shp))
    return pl.pallas_call(
        functools.partial(_proj_body, n_src, n_dst),
        grid=(grid,),
        in_specs=[
            pl.BlockSpec((blk, HID), lambda i: (i, 0)),
            full((HID, HID)),
            full((1, HID)),
            full((HID, n_src * L)),
            full((HID, n_dst * L)),
            full((HID, HID)),
            full((L, 2 * L)),
        ],
        out_specs=[pl.BlockSpec((blk, TW), lambda i: (i, 0))] * n_src
        + [pl.BlockSpec((blk, L), lambda i: (i, 0))] * n_dst,
        out_shape=[jax.ShapeDtypeStruct((N_NODE, TW), jnp.bfloat16)] * n_src
        + [jax.ShapeDtypeStruct((N_NODE, L), jnp.float32)] * n_dst,
    )


def _epi_body(acc_ap_ref, acc_pa_ref, acc_pp_ref, emat_ref, kw_ref, kb_ref,
              out_ap_ref, out_pa_ref, out_pp_ref, sums_ref):
    i = pl.program_id(0)
    outs = []
    for acc_ref, out_ref in ((acc_ap_ref, out_ap_ref),
                             (acc_pa_ref, out_pa_ref),
                             (acc_pp_ref, out_pp_ref)):
        t = acc_ref[0] + acc_ref[1]
        num = t[:, :HID]
        den = jnp.dot(t[:, HID:], emat_ref[...], precision=_HIGH)
        o = jnp.maximum(num / (den + 1e-16), 0.0)
        out_ref[...] = o
        outs.append(o)
    t_ap = jnp.tanh(jnp.dot(outs[0], kw_ref[...], precision=_HIGH)
                    + kb_ref[...])
    t_pp = jnp.tanh(jnp.dot(outs[2], kw_ref[...], precision=_HIGH)
                    + kb_ref[...])
    part = jnp.concatenate([t_ap.sum(0, keepdims=True),
                            t_pp.sum(0, keepdims=True)], axis=0)

    @pl.when(i == 0)
    def _():
        sums_ref[...] = jnp.zeros_like(sums_ref)

    sums_ref[...] += part


def _make_epi():
    blk = 1000
    grid = N_NODE // blk
    acc_spec = pl.BlockSpec((NC, blk, RW), lambda i: (0, i, 0))
    full = lambda shp: pl.BlockSpec(shp, lambda i: (0,) * len(shp))
    return pl.pallas_call(
        _epi_body,
        grid=(grid,),
        in_specs=[acc_spec, acc_spec, acc_spec,
                  full((L, HID)), full((HID, HID)), full((1, HID))],
        out_specs=[pl.BlockSpec((blk, HID), lambda i: (i, 0))] * 3
        + [full((2, HID))],
        out_shape=[jax.ShapeDtypeStruct((N_NODE, HID), jnp.float32)] * 3
        + [jax.ShapeDtypeStruct((2, HID), jnp.float32)],
    )


def _mix_body(x_ref, y_ref, attn_ref, out_ref):
    out_ref[...] = jnp.maximum(
        attn_ref[0] * x_ref[...] + attn_ref[1] * y_ref[...], 0.0)


def _make_mix():
    blk = 1000
    grid = N_NODE // blk
    return pl.pallas_call(
        _mix_body,
        grid=(grid,),
        in_specs=[pl.BlockSpec((blk, HID), lambda i: (i, 0)),
                  pl.BlockSpec((blk, HID), lambda i: (i, 0)),
                  pl.BlockSpec(memory_space=pltpu.SMEM)],
        out_specs=pl.BlockSpec((blk, HID), lambda i: (i, 0)),
        out_shape=jax.ShapeDtypeStruct((N_NODE, HID), jnp.float32),
    )


_proj_a = _make_proj(1, 1)
_proj_p = _make_proj(2, 2)
_epi = _make_epi()
_mix = _make_mix()


def _att_mat(att):
    sel = _np.zeros((HID, L), _np.float32)
    for h in range(HEADS):
        sel[h * D:(h + 1) * D, h] = 1.0
    return jnp.tile(att.reshape(HID)[:, None], (1, L)) * sel


def _emat():
    m = _np.zeros((L, HID), _np.float32)
    for h in range(HEADS):
        m[h, h * D:(h + 1) * D] = 1.0
    return m


def _pmat():
    m = _np.zeros((HID, HID), _np.float32)
    for j in range(HID // (2 * L)):
        for i in range(L):
            m[32 * j + i, 32 * j + 2 * i] = 1.0
            m[32 * j + L + i, 32 * j + 2 * i + 1] = 1.0
    return m


def _smat():
    m = _np.zeros((L, 2 * L), _np.float32)
    for i in range(L):
        m[i, 2 * i] = 1.0
    return m


def _pad_edges(ei):
    npad = EPAD - E
    esrc = jnp.concatenate(
        [ei[0], jnp.zeros((npad,), jnp.int32)]).reshape(NW * NCHUNK, C)
    edst = jnp.concatenate(
        [ei[1], jnp.full((npad,), N_NODE, jnp.int32)]).reshape(NW * NCHUNK, C)
    return esrc, edst


def _layer(x_a, x_p, edges, p):
    (esrc_ap, edst_ap), (esrc_pa, edst_pa), (esrc_pp, edst_pp) = edges
    b_a = p['proj_author_b'].reshape(1, HID)
    b_p = p['proj_paper_b'].reshape(1, HID)
    asrc_a = _att_mat(p['att_src_ap'])
    adst_a = _att_mat(p['att_dst_pa'])
    asrc_p = jnp.concatenate(
        [_att_mat(p['att_src_pa']), _att_mat(p['att_src_pp'])], axis=1)
    adst_p = jnp.concatenate(
        [_att_mat(p['att_dst_ap']), _att_mat(p['att_dst_pp'])], axis=1)

    pmat = jnp.asarray(_pmat())
    smat = jnp.asarray(_smat())
    haug_ap, adst_pa16 = _proj_a(x_a, p['proj_author_W'], b_a, asrc_a, adst_a,
                                 pmat, smat)
    haug_pa, haug_pp, adst_ap16, adst_pp16 = _proj_p(
        x_p, p['proj_paper_W'], b_p, asrc_p, adst_p, pmat, smat)

    def _as_i32(t):
        return lax.bitcast_convert_type(
            t.reshape(N_NODE, TW // 2, 2), jnp.int32)

    haug_ap, haug_pa, haug_pp = map(_as_i32, (haug_ap, haug_pa, haug_pp))

    acc_ap = _edge_pass(haug_ap, adst_ap16, esrc_ap, edst_ap)
    acc_pa = _edge_pass(haug_pa, adst_pa16, esrc_pa, edst_pa)
    acc_pp = _edge_pass(haug_pp, adst_pp16, esrc_pp, edst_pp)

    out_ap, out_pa, out_pp, sums = _epi(
        acc_ap, acc_pa, acc_pp, jnp.asarray(_emat()), p['k_lin_W'],
        p['k_lin_b'].reshape(1, HID))

    score = (p['q'][None, :] * (sums / N_NODE)).sum(-1)
    attn = jax.nn.softmax(score)
    p_out = _mix(out_ap, out_pp, attn)
    return out_pa, p_out


def kernel(x_author, x_paper, edge_index_author_paper,
           edge_index_paper_author, edge_index_paper_paper,
           params1, params2):
    edges = (_pad_edges(edge_index_author_paper),
             _pad_edges(edge_index_paper_author),
             _pad_edges(edge_index_paper_paper))
    a1, p1 = _layer(x_author, x_paper, edges, params1)
    a2, p2 = _layer(a1, p1, edges, params2)
    return a2, p2
